# baseline (speedup 1.0000x reference)
import functools

import jax
import jax.numpy as jnp
from jax import lax
from jax.experimental import pallas as pl
from jax.experimental.pallas import tpu as pltpu
from jax.experimental.pallas import tpu_sc as plsc

T = 2048
E = 8
D = 1024
FF = 2048
TOPK = 2
TILE = 128
MAXTILES = (T * TOPK) // TILE + E
PAD = MAXTILES * TILE
LANES = 128
NEG = -1e30

NC = 2
NS = 16
NW = NC * NS
SCL = 16

A = T * TOPK
DISPATCH_CHUNK = 32
COMBINE_CHUNK = 16


def _gate_kernel(x_ref, mf_ref, wg_ref,
                 logits_ref, p0_ref, p1_ref, eot_ref, w0b_ref, w1b_ref,
                 need_ref, slot_ref, nxte_ref):
    x = x_ref[...]
    gi = x + mf_ref[...]
    logits = jnp.dot(gi, wg_ref[...], preferred_element_type=jnp.float32)
    logits_ref[...] = logits

    lane = lax.broadcasted_iota(jnp.int32, (T, LANES), 1)
    lm = jnp.where(lane < E, logits, NEG)
    m1 = jnp.max(lm, axis=1, keepdims=True)
    i1 = jnp.min(jnp.where(lm == m1, lane, LANES), axis=1, keepdims=True)
    lm2 = jnp.where(lane == i1, NEG, lm)
    m2 = jnp.max(lm2, axis=1, keepdims=True)
    i2 = jnp.min(jnp.where(lm2 == m2, lane, LANES), axis=1, keepdims=True)

    r = jnp.exp(m2 - m1)
    w0 = 1.0 / (1.0 + r)
    w1 = r / (1.0 + r)
    w0b_ref[...] = jnp.broadcast_to(w0, (T, SCL))
    w1b_ref[...] = jnp.broadcast_to(w1, (T, SCL))

    oh0 = (lane == i1).astype(jnp.float32)
    oh1 = (lane == i2).astype(jnp.float32)
    tr = lax.broadcasted_iota(jnp.int32, (T, T), 0)
    tc = lax.broadcasted_iota(jnp.int32, (T, T), 1)
    ls = (tc < tr).astype(jnp.bfloat16)
    pre0 = jnp.dot(ls, oh0.astype(jnp.bfloat16), preferred_element_type=jnp.float32)
    pre1 = jnp.dot(ls, oh1.astype(jnp.bfloat16), preferred_element_type=jnp.float32)
    cnt0 = jnp.sum(oh0, axis=0, keepdims=True)
    cnt1 = jnp.sum(oh1, axis=0, keepdims=True)
    counts = cnt0 + cnt1
    rank0 = jnp.sum(oh0 * pre0, axis=1, keepdims=True)
    rank1 = jnp.sum(oh1 * (pre1 + cnt0), axis=1, keepdims=True)

    tiles = jnp.floor((counts + (TILE - 1)) / TILE)
    slots = tiles * TILE
    lr = lax.broadcasted_iota(jnp.int32, (LANES, LANES), 0)
    lc = lax.broadcasted_iota(jnp.int32, (LANES, LANES), 1)
    um = (lr < lc).astype(jnp.float32)
    off = jnp.dot(slots, um, preferred_element_type=jnp.float32)
    og0 = jnp.sum(oh0 * off, axis=1, keepdims=True)
    og1 = jnp.sum(oh1 * off, axis=1, keepdims=True)
    p0_ref[...] = (rank0 + og0).astype(jnp.int32)
    p1_ref[...] = (rank1 + og1).astype(jnp.int32)

    um_le = (lr <= lc).astype(jnp.float32)
    cumt = jnp.dot(tiles, um_le, preferred_element_type=jnp.float32)
    cum_b = jnp.broadcast_to(cumt, (LANES, LANES))
    i_row = lax.broadcasted_iota(jnp.int32, (LANES, LANES), 0).astype(jnp.float32)
    contrib = jnp.where((lc < E) & (cum_b <= i_row), 1.0, 0.0)
    eot = jnp.minimum(jnp.sum(contrib, axis=1, keepdims=True), float(E - 1))
    eot_i = eot.astype(jnp.int32)
    eot_ref[...] = eot_i

    sh = jnp.concatenate(
        [jnp.full((1, 1), -1, jnp.int32), eot_i[:LANES - 1]], axis=0)
    need = (eot_i != sh).astype(jnp.int32)
    tril_incl = (lr >= lc).astype(jnp.float32)
    cums = jnp.dot(tril_incl, need.astype(jnp.float32),
                   preferred_element_type=jnp.float32)
    seg = cums.astype(jnp.int32) - 1
    slot = seg - 2 * (seg // 2)
    need_ref[...] = need
    slot_ref[...] = slot
    active = counts > 0.0
    cand = jnp.where((lc > jnp.broadcast_to(eot_i, (LANES, LANES)))
                     & (lc < E) & jnp.broadcast_to(active, (LANES, LANES)),
                     lc, LANES)
    nxt = jnp.min(cand, axis=1, keepdims=True)
    nxte_ref[...] = jnp.minimum(nxt, E)


def _gate_call(x, mf, wgp):
    return pl.pallas_call(
        _gate_kernel,
        out_shape=(
            jax.ShapeDtypeStruct((T, LANES), jnp.float32),
            jax.ShapeDtypeStruct((T, 1), jnp.int32),
            jax.ShapeDtypeStruct((T, 1), jnp.int32),
            jax.ShapeDtypeStruct((LANES, 1), jnp.int32),
            jax.ShapeDtypeStruct((T, SCL), jnp.float32),
            jax.ShapeDtypeStruct((T, SCL), jnp.float32),
            jax.ShapeDtypeStruct((LANES, 1), jnp.int32),
            jax.ShapeDtypeStruct((LANES, 1), jnp.int32),
            jax.ShapeDtypeStruct((LANES, 1), jnp.int32),
        ),
        compiler_params=pltpu.CompilerParams(
            vmem_limit_bytes=120 * 1024 * 1024),
    )(x, mf, wgp)


def _make_dispatch():
    a_per_w = A // NW
    n_ch = a_per_w // DISPATCH_CHUNK
    t_per_w = a_per_w

    mesh = plsc.VectorSubcoreMesh(core_axis_name="c", subcore_axis_name="s")

    @functools.partial(
        pl.kernel, mesh=mesh,
        out_type=jax.ShapeDtypeStruct((PAD, D), jnp.float32),
        scratch_types=[
            pltpu.VMEM((DISPATCH_CHUNK,), jnp.int32),
            pltpu.VMEM((DISPATCH_CHUNK,), jnp.int32),
            pltpu.VMEM((DISPATCH_CHUNK, D), jnp.float32),
            pltpu.VMEM((DISPATCH_CHUNK, D), jnp.float32),
            pltpu.SemaphoreType.DMA,
            pltpu.SemaphoreType.DMA,
        ],
    )
    def dispatch(x_hbm, p_hbm, xs_hbm, idx_a, idx_b, rows_a, rows_b,
                 sem_a, sem_b):
        wid = lax.axis_index("s") * NC + lax.axis_index("c")
        a_base = wid * a_per_w
        t_base = (wid % NS) * t_per_w
        bufs = (rows_a, rows_b)
        idxs = (idx_a, idx_b)
        sems = (sem_a, sem_b)
        scatters = {}
        for c in range(n_ch):
            buf = bufs[c % 2]
            idx = idxs[c % 2]
            sem = sems[c % 2]
            if c >= 2:
                scatters[c - 2].wait()
            pltpu.sync_copy(
                x_hbm.at[pl.ds(t_base + c * DISPATCH_CHUNK, DISPATCH_CHUNK)],
                buf)
            pltpu.sync_copy(
                p_hbm.at[pl.ds(a_base + c * DISPATCH_CHUNK, DISPATCH_CHUNK)],
                idx)
            scatters[c] = pltpu.async_copy(buf, xs_hbm.at[idx], sem)
        scatters[n_ch - 2].wait()
        scatters[n_ch - 1].wait()

    return dispatch


_dispatch = _make_dispatch()


def _expert_kernel(eot_s, need_s, slot_s, nxte_s, xs_ref,
                   w1_hbm, v1_hbm, w2_hbm, y_ref,
                   w1b0, v1b0, w2b0, w1b1, v1b1, w2b1, sem):
    i = pl.program_id(0)
    e = eot_s[i]
    slot = slot_s[i]
    bufs = ((w1b0, v1b0, w2b0), (w1b1, v1b1, w2b1))
    hbms = (w1_hbm, v1_hbm, w2_hbm)

    @pl.when(i == 0)
    def _():
        for hbm, buf in zip(hbms, bufs[0]):
            pltpu.make_async_copy(hbm.at[e], buf, sem).start()

    for s in (0, 1):
        @pl.when((need_s[i] == 1) & (slot == s))
        def _(s=s):
            for hbm, buf in zip(hbms, bufs[s]):
                pltpu.make_async_copy(hbm.at[e], buf, sem).wait()
            ne = nxte_s[i]

            @pl.when(ne < E)
            def _():
                for hbm, buf in zip(hbms, bufs[1 - s]):
                    pltpu.make_async_copy(hbm.at[ne], buf, sem).start()

    xg = xs_ref[...].astype(jnp.bfloat16)

    for s in (0, 1):
        @pl.when(slot == s)
        def _(s=s):
            w1b, v1b, w2b = bufs[s]
            h = jnp.dot(xg, w1b[...].astype(jnp.bfloat16),
                        preferred_element_type=jnp.float32)
            g = jnp.dot(xg, v1b[...].astype(jnp.bfloat16),
                        preferred_element_type=jnp.float32)
            u = (0.5 * h * (1.0 + lax.erf(h * (2.0 ** -0.5)))) * g
            y = jnp.dot(u.astype(jnp.bfloat16), w2b[...].astype(jnp.bfloat16),
                        preferred_element_type=jnp.float32)
            y_ref[...] = y


def _expert_call(eot, need, slot, nxte, xs, W1, V1, W2):
    grid_spec = pltpu.PrefetchScalarGridSpec(
        num_scalar_prefetch=4,
        grid=(MAXTILES,),
        in_specs=[
            pl.BlockSpec((TILE, D), lambda i, *_: (i, 0)),
            pl.BlockSpec(memory_space=pl.ANY),
            pl.BlockSpec(memory_space=pl.ANY),
            pl.BlockSpec(memory_space=pl.ANY),
        ],
        out_specs=pl.BlockSpec((TILE, D), lambda i, *_: (i, 0)),
        scratch_shapes=[
            pltpu.VMEM((D, FF), jnp.float32),
            pltpu.VMEM((D, FF), jnp.float32),
            pltpu.VMEM((FF, D), jnp.float32),
            pltpu.VMEM((D, FF), jnp.float32),
            pltpu.VMEM((D, FF), jnp.float32),
            pltpu.VMEM((FF, D), jnp.float32),
            pltpu.SemaphoreType.DMA,
        ],
    )
    return pl.pallas_call(
        _expert_kernel,
        grid_spec=grid_spec,
        out_shape=jax.ShapeDtypeStruct((PAD, D), jnp.float32),
        compiler_params=pltpu.CompilerParams(
            vmem_limit_bytes=120 * 1024 * 1024),
    )(eot, need, slot, nxte, xs, W1, V1, W2)


def _make_combine():
    tok_per_w = T // NW
    n_ch = tok_per_w // COMBINE_CHUNK
    vchunks = D // SCL

    mesh = plsc.VectorSubcoreMesh(core_axis_name="c", subcore_axis_name="s")

    @functools.partial(
        pl.kernel, mesh=mesh,
        out_type=jax.ShapeDtypeStruct((T, D), jnp.float32),
        scratch_types=[
            pltpu.VMEM((2, COMBINE_CHUNK), jnp.int32),
            pltpu.VMEM((2, COMBINE_CHUNK), jnp.int32),
            pltpu.VMEM((2, COMBINE_CHUNK, SCL), jnp.float32),
            pltpu.VMEM((2, COMBINE_CHUNK, SCL), jnp.float32),
            pltpu.VMEM((2, COMBINE_CHUNK, D), jnp.float32),
            pltpu.VMEM((2, COMBINE_CHUNK, D), jnp.float32),
            pltpu.VMEM((COMBINE_CHUNK, D), jnp.float32),
            pltpu.SemaphoreType.DMA,
            pltpu.SemaphoreType.DMA,
            pltpu.SemaphoreType.DMA,
            pltpu.SemaphoreType.DMA,
        ],
    )
    def combine(y_hbm, p0_hbm, p1_hbm, w0b_hbm, w1b_hbm, out_hbm,
                i0_v, i1_v, w0_v, w1_v, r0_v, r1_v, o_v,
                sem0a, sem1a, sem0b, sem1b):
        wid = lax.axis_index("s") * NC + lax.axis_index("c")
        base = wid * tok_per_w
        sems = ((sem0a, sem1a), (sem0b, sem1b))

        def start(c):
            par = c % 2
            b = base + c * COMBINE_CHUNK
            pltpu.sync_copy(p0_hbm.at[pl.ds(b, COMBINE_CHUNK)], i0_v.at[par])
            pltpu.sync_copy(p1_hbm.at[pl.ds(b, COMBINE_CHUNK)], i1_v.at[par])
            pltpu.sync_copy(w0b_hbm.at[pl.ds(b, COMBINE_CHUNK)], w0_v.at[par])
            pltpu.sync_copy(w1b_hbm.at[pl.ds(b, COMBINE_CHUNK)], w1_v.at[par])
            c0 = pltpu.async_copy(y_hbm.at[i0_v.at[par]], r0_v.at[par],
                                  sems[par][0])
            c1 = pltpu.async_copy(y_hbm.at[i1_v.at[par]], r1_v.at[par],
                                  sems[par][1])
            return (c0, c1)

        pending = {0: start(0)}
        for c in range(n_ch):
            par = c % 2
            if c + 1 < n_ch:
                pending[c + 1] = start(c + 1)
            pending[c][0].wait()
            pending[c][1].wait()

            def row_body(j, carry):
                wa = w0_v[par, j, :]
                wb = w1_v[par, j, :]
                for v in range(vchunks):
                    sl = pl.ds(v * SCL, SCL)
                    o_v[j, sl] = r0_v[par, j, sl] * wa + r1_v[par, j, sl] * wb
                return carry

            lax.fori_loop(0, COMBINE_CHUNK, row_body, 0)
            pltpu.sync_copy(o_v, out_hbm.at[pl.ds(base + c * COMBINE_CHUNK,
                                                  COMBINE_CHUNK)])

    return combine


_combine = _make_combine()


def kernel(hidden_states, mod_feat, Wg, W1, V1, W2):
    b, s, d = hidden_states.shape
    x = hidden_states.reshape(T, D)
    mf = mod_feat.reshape(1, D)
    wgp = jnp.pad(Wg, ((0, 0), (0, LANES - E)))

    logits128, p0, p1, eot, w0b, w1b, need, slot, nxte = _gate_call(x, mf, wgp)
    router_logits = logits128[:, :E]
    p_cat = jnp.concatenate([p0.reshape(T), p1.reshape(T)])
    eot_flat = eot.reshape(LANES)[:MAXTILES]
    need_flat = need.reshape(LANES)[:MAXTILES]
    slot_flat = slot.reshape(LANES)[:MAXTILES]
    nxte_flat = nxte.reshape(LANES)[:MAXTILES]

    xs = _dispatch(x, p_cat)
    y = _expert_call(eot_flat, need_flat, slot_flat, nxte_flat, xs, W1, V1, W2)
    out = _combine(y, p0.reshape(T), p1.reshape(T), w0b, w1b)
    return out.reshape(b, s, d), router_logits


# speedup vs baseline: 1.0084x; 1.0084x over previous
import functools

import jax
import jax.numpy as jnp
from jax import lax
from jax.experimental import pallas as pl
from jax.experimental.pallas import tpu as pltpu
from jax.experimental.pallas import tpu_sc as plsc

T = 2048
E = 8
D = 1024
FF = 2048
TOPK = 2
TILE = 128
MAXTILES = (T * TOPK) // TILE + E
PAD = MAXTILES * TILE
LANES = 128
NEG = -1e30

NC = 2
NS = 16
NW = NC * NS
SCL = 16

A = T * TOPK
DISPATCH_CHUNK = 32
COMBINE_CHUNK = 16


def _gate_kernel(x_ref, mf_ref, wg_ref,
                 logits_ref, p0_ref, p1_ref, eot_ref, w0b_ref, w1b_ref,
                 need_ref, slot_ref, nxte_ref):
    x = x_ref[...]
    gi = x + mf_ref[...]
    logits = jnp.dot(gi, wg_ref[...], preferred_element_type=jnp.float32)
    logits_ref[...] = logits

    lane = lax.broadcasted_iota(jnp.int32, (T, LANES), 1)
    lm = jnp.where(lane < E, logits, NEG)
    m1 = jnp.max(lm, axis=1, keepdims=True)
    i1 = jnp.min(jnp.where(lm == m1, lane, LANES), axis=1, keepdims=True)
    lm2 = jnp.where(lane == i1, NEG, lm)
    m2 = jnp.max(lm2, axis=1, keepdims=True)
    i2 = jnp.min(jnp.where(lm2 == m2, lane, LANES), axis=1, keepdims=True)

    r = jnp.exp(m2 - m1)
    w0 = 1.0 / (1.0 + r)
    w1 = r / (1.0 + r)
    w0b_ref[...] = jnp.broadcast_to(w0, (T, SCL))
    w1b_ref[...] = jnp.broadcast_to(w1, (T, SCL))

    oh0 = (lane == i1).astype(jnp.float32)
    oh1 = (lane == i2).astype(jnp.float32)
    tr = lax.broadcasted_iota(jnp.int32, (T, T), 0)
    tc = lax.broadcasted_iota(jnp.int32, (T, T), 1)
    ls = (tc < tr).astype(jnp.bfloat16)
    pre0 = jnp.dot(ls, oh0.astype(jnp.bfloat16), preferred_element_type=jnp.float32)
    pre1 = jnp.dot(ls, oh1.astype(jnp.bfloat16), preferred_element_type=jnp.float32)
    cnt0 = jnp.sum(oh0, axis=0, keepdims=True)
    cnt1 = jnp.sum(oh1, axis=0, keepdims=True)
    counts = cnt0 + cnt1
    rank0 = jnp.sum(oh0 * pre0, axis=1, keepdims=True)
    rank1 = jnp.sum(oh1 * (pre1 + cnt0), axis=1, keepdims=True)

    tiles = jnp.floor((counts + (TILE - 1)) / TILE)
    slots = tiles * TILE
    lr = lax.broadcasted_iota(jnp.int32, (LANES, LANES), 0)
    lc = lax.broadcasted_iota(jnp.int32, (LANES, LANES), 1)
    um = (lr < lc).astype(jnp.float32)
    off = jnp.dot(slots, um, preferred_element_type=jnp.float32)
    og0 = jnp.sum(oh0 * off, axis=1, keepdims=True)
    og1 = jnp.sum(oh1 * off, axis=1, keepdims=True)
    p0_ref[...] = (rank0 + og0).astype(jnp.int32)
    p1_ref[...] = (rank1 + og1).astype(jnp.int32)

    um_le = (lr <= lc).astype(jnp.float32)
    cumt = jnp.dot(tiles, um_le, preferred_element_type=jnp.float32)
    cum_b = jnp.broadcast_to(cumt, (LANES, LANES))
    i_row = lax.broadcasted_iota(jnp.int32, (LANES, LANES), 0).astype(jnp.float32)
    contrib = jnp.where((lc < E) & (cum_b <= i_row), 1.0, 0.0)
    eot = jnp.minimum(jnp.sum(contrib, axis=1, keepdims=True), float(E - 1))
    eot_i = eot.astype(jnp.int32)
    eot_ref[...] = eot_i

    sh = jnp.concatenate(
        [jnp.full((1, 1), -1, jnp.int32), eot_i[:LANES - 1]], axis=0)
    need = (eot_i != sh).astype(jnp.int32)
    tril_incl = (lr >= lc).astype(jnp.float32)
    cums = jnp.dot(tril_incl, need.astype(jnp.float32),
                   preferred_element_type=jnp.float32)
    seg = cums.astype(jnp.int32) - 1
    slot = seg - 2 * (seg // 2)
    need_ref[...] = need
    slot_ref[...] = slot
    active = counts > 0.0
    cand = jnp.where((lc > jnp.broadcast_to(eot_i, (LANES, LANES)))
                     & (lc < E) & jnp.broadcast_to(active, (LANES, LANES)),
                     lc, LANES)
    nxt = jnp.min(cand, axis=1, keepdims=True)
    nxte_ref[...] = jnp.minimum(nxt, E)


def _gate_call(x, mf, wgp):
    return pl.pallas_call(
        _gate_kernel,
        out_shape=(
            jax.ShapeDtypeStruct((T, LANES), jnp.float32),
            jax.ShapeDtypeStruct((T, 1), jnp.int32),
            jax.ShapeDtypeStruct((T, 1), jnp.int32),
            jax.ShapeDtypeStruct((LANES, 1), jnp.int32),
            jax.ShapeDtypeStruct((T, SCL), jnp.float32),
            jax.ShapeDtypeStruct((T, SCL), jnp.float32),
            jax.ShapeDtypeStruct((LANES, 1), jnp.int32),
            jax.ShapeDtypeStruct((LANES, 1), jnp.int32),
            jax.ShapeDtypeStruct((LANES, 1), jnp.int32),
        ),
        compiler_params=pltpu.CompilerParams(
            vmem_limit_bytes=120 * 1024 * 1024),
    )(x, mf, wgp)


def _make_dispatch():
    a_per_w = A // NW
    n_ch = a_per_w // DISPATCH_CHUNK
    t_per_w = a_per_w

    mesh = plsc.VectorSubcoreMesh(core_axis_name="c", subcore_axis_name="s")

    @functools.partial(
        pl.kernel, mesh=mesh,
        out_type=jax.ShapeDtypeStruct((PAD, D), jnp.float32),
        scratch_types=[
            pltpu.VMEM((DISPATCH_CHUNK,), jnp.int32),
            pltpu.VMEM((DISPATCH_CHUNK,), jnp.int32),
            pltpu.VMEM((DISPATCH_CHUNK, D), jnp.float32),
            pltpu.VMEM((DISPATCH_CHUNK, D), jnp.float32),
            pltpu.SemaphoreType.DMA,
            pltpu.SemaphoreType.DMA,
        ],
    )
    def dispatch(x_hbm, p_hbm, xs_hbm, idx_a, idx_b, rows_a, rows_b,
                 sem_a, sem_b):
        wid = lax.axis_index("s") * NC + lax.axis_index("c")
        a_base = wid * a_per_w
        t_base = (wid % NS) * t_per_w
        bufs = (rows_a, rows_b)
        idxs = (idx_a, idx_b)
        sems = (sem_a, sem_b)
        scatters = {}
        for c in range(n_ch):
            buf = bufs[c % 2]
            idx = idxs[c % 2]
            sem = sems[c % 2]
            if c >= 2:
                scatters[c - 2].wait()
            pltpu.sync_copy(
                x_hbm.at[pl.ds(t_base + c * DISPATCH_CHUNK, DISPATCH_CHUNK)],
                buf)
            pltpu.sync_copy(
                p_hbm.at[pl.ds(a_base + c * DISPATCH_CHUNK, DISPATCH_CHUNK)],
                idx)
            scatters[c] = pltpu.async_copy(buf, xs_hbm.at[idx], sem)
        scatters[n_ch - 2].wait()
        scatters[n_ch - 1].wait()

    return dispatch


_dispatch = _make_dispatch()


def _expert_kernel(eot_s, need_s, slot_s, nxte_s, xs_ref,
                   w1_hbm, v1_hbm, w2_hbm, y_ref,
                   w1b, v1b, w2b, sem):
    i = pl.program_id(0)
    e = eot_s[i]
    slot = slot_s[i]

    @pl.when(i == 0)
    def _():
        pltpu.make_async_copy(w1_hbm.at[e], w1b.at[0], sem).start()
        pltpu.make_async_copy(v1_hbm.at[e], v1b.at[0], sem).start()
        pltpu.make_async_copy(w2_hbm.at[e], w2b.at[0], sem).start()

    @pl.when(need_s[i] == 1)
    def _():
        pltpu.make_async_copy(w1_hbm.at[e], w1b.at[slot], sem).wait()
        pltpu.make_async_copy(v1_hbm.at[e], v1b.at[slot], sem).wait()
        pltpu.make_async_copy(w2_hbm.at[e], w2b.at[slot], sem).wait()
        ne = nxte_s[i]

        @pl.when(ne < E)
        def _():
            pltpu.make_async_copy(w1_hbm.at[ne], w1b.at[1 - slot], sem).start()
            pltpu.make_async_copy(v1_hbm.at[ne], v1b.at[1 - slot], sem).start()
            pltpu.make_async_copy(w2_hbm.at[ne], w2b.at[1 - slot], sem).start()

    xg = xs_ref[...].astype(jnp.bfloat16)
    h = jnp.dot(xg, w1b[slot].astype(jnp.bfloat16),
                preferred_element_type=jnp.float32)
    g = jnp.dot(xg, v1b[slot].astype(jnp.bfloat16),
                preferred_element_type=jnp.float32)
    u = (0.5 * h * (1.0 + lax.erf(h * (2.0 ** -0.5)))) * g
    y = jnp.dot(u.astype(jnp.bfloat16), w2b[slot].astype(jnp.bfloat16),
                preferred_element_type=jnp.float32)
    y_ref[...] = y


def _expert_call(eot, need, slot, nxte, xs, W1, V1, W2):
    grid_spec = pltpu.PrefetchScalarGridSpec(
        num_scalar_prefetch=4,
        grid=(MAXTILES,),
        in_specs=[
            pl.BlockSpec((TILE, D), lambda i, *_: (i, 0)),
            pl.BlockSpec(memory_space=pl.ANY),
            pl.BlockSpec(memory_space=pl.ANY),
            pl.BlockSpec(memory_space=pl.ANY),
        ],
        out_specs=pl.BlockSpec((TILE, D), lambda i, *_: (i, 0)),
        scratch_shapes=[
            pltpu.VMEM((2, D, FF), jnp.float32),
            pltpu.VMEM((2, D, FF), jnp.float32),
            pltpu.VMEM((2, FF, D), jnp.float32),
            pltpu.SemaphoreType.DMA,
        ],
    )
    return pl.pallas_call(
        _expert_kernel,
        grid_spec=grid_spec,
        out_shape=jax.ShapeDtypeStruct((PAD, D), jnp.float32),
        compiler_params=pltpu.CompilerParams(
            vmem_limit_bytes=120 * 1024 * 1024),
    )(eot, need, slot, nxte, xs, W1, V1, W2)


def _make_combine():
    tok_per_w = T // NW
    n_ch = tok_per_w // COMBINE_CHUNK
    vchunks = D // SCL

    mesh = plsc.VectorSubcoreMesh(core_axis_name="c", subcore_axis_name="s")

    @functools.partial(
        pl.kernel, mesh=mesh,
        out_type=jax.ShapeDtypeStruct((T, D), jnp.float32),
        scratch_types=[
            pltpu.VMEM((2, COMBINE_CHUNK), jnp.int32),
            pltpu.VMEM((2, COMBINE_CHUNK), jnp.int32),
            pltpu.VMEM((2, COMBINE_CHUNK, SCL), jnp.float32),
            pltpu.VMEM((2, COMBINE_CHUNK, SCL), jnp.float32),
            pltpu.VMEM((2, COMBINE_CHUNK, D), jnp.float32),
            pltpu.VMEM((2, COMBINE_CHUNK, D), jnp.float32),
            pltpu.VMEM((COMBINE_CHUNK, D), jnp.float32),
            pltpu.SemaphoreType.DMA,
            pltpu.SemaphoreType.DMA,
            pltpu.SemaphoreType.DMA,
            pltpu.SemaphoreType.DMA,
        ],
    )
    def combine(y_hbm, p0_hbm, p1_hbm, w0b_hbm, w1b_hbm, out_hbm,
                i0_v, i1_v, w0_v, w1_v, r0_v, r1_v, o_v,
                sem0a, sem1a, sem0b, sem1b):
        wid = lax.axis_index("s") * NC + lax.axis_index("c")
        base = wid * tok_per_w
        sems = ((sem0a, sem1a), (sem0b, sem1b))

        def start(c):
            par = c % 2
            b = base + c * COMBINE_CHUNK
            pltpu.sync_copy(p0_hbm.at[pl.ds(b, COMBINE_CHUNK)], i0_v.at[par])
            pltpu.sync_copy(p1_hbm.at[pl.ds(b, COMBINE_CHUNK)], i1_v.at[par])
            pltpu.sync_copy(w0b_hbm.at[pl.ds(b, COMBINE_CHUNK)], w0_v.at[par])
            pltpu.sync_copy(w1b_hbm.at[pl.ds(b, COMBINE_CHUNK)], w1_v.at[par])
            c0 = pltpu.async_copy(y_hbm.at[i0_v.at[par]], r0_v.at[par],
                                  sems[par][0])
            c1 = pltpu.async_copy(y_hbm.at[i1_v.at[par]], r1_v.at[par],
                                  sems[par][1])
            return (c0, c1)

        pending = {0: start(0)}
        for c in range(n_ch):
            par = c % 2
            if c + 1 < n_ch:
                pending[c + 1] = start(c + 1)
            pending[c][0].wait()
            pending[c][1].wait()

            def row_body(j, carry):
                wa = w0_v[par, j, :]
                wb = w1_v[par, j, :]
                for v in range(vchunks):
                    sl = pl.ds(v * SCL, SCL)
                    o_v[j, sl] = r0_v[par, j, sl] * wa + r1_v[par, j, sl] * wb
                return carry

            lax.fori_loop(0, COMBINE_CHUNK, row_body, 0)
            pltpu.sync_copy(o_v, out_hbm.at[pl.ds(base + c * COMBINE_CHUNK,
                                                  COMBINE_CHUNK)])

    return combine


_combine = _make_combine()


def kernel(hidden_states, mod_feat, Wg, W1, V1, W2):
    b, s, d = hidden_states.shape
    x = hidden_states.reshape(T, D)
    mf = mod_feat.reshape(1, D)
    wgp = jnp.pad(Wg, ((0, 0), (0, LANES - E)))

    logits128, p0, p1, eot, w0b, w1b, need, slot, nxte = _gate_call(x, mf, wgp)
    router_logits = logits128[:, :E]
    p_cat = jnp.concatenate([p0.reshape(T), p1.reshape(T)])
    eot_flat = eot.reshape(LANES)[:MAXTILES]
    need_flat = need.reshape(LANES)[:MAXTILES]
    slot_flat = slot.reshape(LANES)[:MAXTILES]
    nxte_flat = nxte.reshape(LANES)[:MAXTILES]

    xs = _dispatch(x, p_cat)
    y = _expert_call(eot_flat, need_flat, slot_flat, nxte_flat, xs, W1, V1, W2)
    out = _combine(y, p0.reshape(T), p1.reshape(T), w0b, w1b)
    return out.reshape(b, s, d), router_logits
